# Initial kernel scaffold; baseline (speedup 1.0000x reference)
import jax
import jax.numpy as jnp
from jax.experimental import pallas as pl


def kernel(x, edge_index, batch, a2_src, a2_dst, iso_type_2, edge_index_2, batch_2, a3_src, a3_dst, iso_type_3, edge_index_3, batch_3, Wr1, Wn1, bc1, Wr2, Wn2, bc2, Wr3, Wn3, bc3, Wr4, Wn4, bc4, Wr5, Wn5, bc5, Wr6, Wn6, bc6, Wr7, Wn7, bc7, fc1_W, fc1_b, fc2_W, fc2_b, fc3_W, fc3_b):
    raise NotImplementedError("write your pallas kernel here")


# speedup vs baseline: 6.7301x; 6.7301x over previous
import functools

import jax
import jax.numpy as jnp
from jax import lax
from jax.experimental import pallas as pl
from jax.experimental.pallas import tpu as pltpu
from jax.experimental.pallas import tpu_sc as plsc

F32 = jnp.float32
N1_, N2_, N3_, NB = 10000, 20000, 15000, 64
NC, NS = 2, 16
NW = NC * NS
CH = 128
BN = 1000


def _ceil(a, b):
    return -(-a // b)


def _elu(x):
    return jnp.where(x > 0, x, jnp.exp(x) - 1.0)


@functools.lru_cache(maxsize=None)
def _segsum_kernel(K, W, NP, with_counts):
    RPT = NP // NS
    mesh = plsc.VectorSubcoreMesh(core_axis_name="c", subcore_axis_name="s")
    outs = [jax.ShapeDtypeStruct((NC, NP, W), F32)]
    scratch = [
        pltpu.VMEM_SHARED((NP, W), F32),
        pltpu.VMEM((K, CH), jnp.int32),
        pltpu.VMEM((K, CH), jnp.int32),
        pltpu.VMEM((CH, W), F32),
        pltpu.SemaphoreType.DMA,
    ]
    if with_counts:
        outs.append(jax.ShapeDtypeStruct((NC, NP, 16), F32))
        scratch += [
            pltpu.VMEM_SHARED((NP, 16), F32),
            pltpu.VMEM((CH, 16), F32),
        ]

    def body(*refs):
        if with_counts:
            (y, srcm, dstm, zrow, zrow_c, ones_h, out, out_c,
             acc, sv, dv, rows, sem, acc_c, onesb) = refs
        else:
            (y, srcm, dstm, zrow, out, acc, sv, dv, rows, sem) = refs
        cid = lax.axis_index("c")
        sid = lax.axis_index("s")
        wid = sid * NC + cid
        pltpu.sync_copy(zrow, acc.at[pl.ds(sid * RPT, RPT)])
        if with_counts:
            pltpu.sync_copy(zrow_c, acc_c.at[pl.ds(sid * RPT, RPT)])
            pltpu.sync_copy(ones_h, onesb)
        pltpu.sync_copy(srcm.at[wid], sv)
        pltpu.sync_copy(dstm.at[wid], dv)
        plsc.subcore_barrier()

        def chunk(j, carry):
            pltpu.async_copy(y.at[sv.at[j]], rows, sem).wait()
            pltpu.sync_copy(rows, acc.at[dv.at[j]], add=True)
            if with_counts:
                pltpu.sync_copy(onesb, acc_c.at[dv.at[j]], add=True)
            return carry

        lax.fori_loop(0, K, chunk, 0)
        plsc.subcore_barrier()
        pltpu.sync_copy(acc.at[pl.ds(sid * RPT, RPT)],
                        out.at[cid, pl.ds(sid * RPT, RPT)])
        if with_counts:
            pltpu.sync_copy(acc_c.at[pl.ds(sid * RPT, RPT)],
                            out_c.at[cid, pl.ds(sid * RPT, RPT)])

    return pl.kernel(
        body, out_type=outs, mesh=mesh, scratch_types=scratch,
        compiler_params=pltpu.CompilerParams(use_tc_tiling_on_sc=False))


def _segsum(y, src, dst, nd, with_counts=False):
    E = src.shape[0]
    W = y.shape[1]
    K = _ceil(E, NW * CH)
    Ep = K * NW * CH
    NP = _ceil(nd + 8, 128) * 128
    pad = Ep - E
    if pad:
        src = jnp.concatenate([src, jnp.zeros((pad,), jnp.int32)])
        dst = jnp.concatenate([dst, jnp.full((pad,), nd, jnp.int32)])
    srcm = src.reshape(NW, K, CH)
    dstm = dst.reshape(NW, K, CH)
    zrow = jnp.zeros((NP // NS, W), F32)
    fn = _segsum_kernel(K, W, NP, with_counts)
    if with_counts:
        zc = jnp.zeros((NP // NS, 16), F32)
        ones_h = jnp.ones((CH, 16), F32)
        return fn(y, srcm, dstm, zrow, zc, ones_h)
    return fn(y, srcm, dstm, zrow)


def _mm2(x, Wr, Wn, b):
    N, Din = x.shape
    Do = Wr.shape[1]

    def body(x_ref, wr_ref, wn_ref, b_ref, r_ref, y_ref):
        xb = x_ref[...]
        r_ref[...] = jnp.dot(xb, wr_ref[...],
                             preferred_element_type=F32) + b_ref[...]
        y_ref[...] = jnp.dot(xb, wn_ref[...], preferred_element_type=F32)

    return pl.pallas_call(
        body, grid=(N // BN,),
        in_specs=[
            pl.BlockSpec((BN, Din), lambda i: (i, 0)),
            pl.BlockSpec((Din, Do), lambda i: (0, 0)),
            pl.BlockSpec((Din, Do), lambda i: (0, 0)),
            pl.BlockSpec((1, Do), lambda i: (0, 0)),
        ],
        out_specs=[pl.BlockSpec((BN, Do), lambda i: (i, 0))] * 2,
        out_shape=[jax.ShapeDtypeStruct((N, Do), F32)] * 2,
    )(x, Wr, Wn, b.reshape(1, -1))


def _mid(r, P, Wr, Wn, b):
    N, Din = r.shape
    NP = P.shape[1]
    Do = Wr.shape[1]

    def body(r_ref, p_ref, wr_ref, wn_ref, b_ref, r2_ref, y2_ref):
        h = _elu(r_ref[...] + p_ref[0] + p_ref[1])
        r2_ref[...] = jnp.dot(h, wr_ref[...],
                              preferred_element_type=F32) + b_ref[...]
        y2_ref[...] = jnp.dot(h, wn_ref[...], preferred_element_type=F32)

    return pl.pallas_call(
        body, grid=(N // BN,),
        in_specs=[
            pl.BlockSpec((BN, Din), lambda i: (i, 0)),
            pl.BlockSpec((NC, BN, Din), lambda i: (0, i, 0)),
            pl.BlockSpec((Din, Do), lambda i: (0, 0)),
            pl.BlockSpec((Din, Do), lambda i: (0, 0)),
            pl.BlockSpec((1, Do), lambda i: (0, 0)),
        ],
        out_specs=[pl.BlockSpec((BN, Do), lambda i: (i, 0))] * 2,
        out_shape=[jax.ShapeDtypeStruct((N, Do), F32)] * 2,
    )(r, P, Wr, Wn, b.reshape(1, -1))


def _last_bpool(r, P, batch3d):
    N, Do = r.shape

    def body(r_ref, p_ref, b_ref, h_ref, s_ref, c_ref):
        h = _elu(r_ref[...] + p_ref[0] + p_ref[1])
        h_ref[...] = h
        bv = b_ref[0, 0, :]
        oh = (bv[None, :] ==
              lax.broadcasted_iota(jnp.int32, (NB, BN), 0)).astype(F32)

        @pl.when(pl.program_id(0) == 0)
        def _():
            s_ref[...] = jnp.zeros_like(s_ref)
            c_ref[...] = jnp.zeros_like(c_ref)

        s_ref[...] += jnp.dot(oh, h, preferred_element_type=F32)
        c_ref[...] += jnp.broadcast_to(
            jnp.sum(oh, axis=1, keepdims=True), (NB, 128))

    return pl.pallas_call(
        body, grid=(N // BN,),
        in_specs=[
            pl.BlockSpec((BN, Do), lambda i: (i, 0)),
            pl.BlockSpec((NC, BN, Do), lambda i: (0, i, 0)),
            pl.BlockSpec((1, 1, BN), lambda i: (i, 0, 0)),
        ],
        out_specs=[
            pl.BlockSpec((BN, Do), lambda i: (i, 0)),
            pl.BlockSpec((NB, Do), lambda i: (0, 0)),
            pl.BlockSpec((NB, 128), lambda i: (0, 0)),
        ],
        out_shape=[
            jax.ShapeDtypeStruct((N, Do), F32),
            jax.ShapeDtypeStruct((NB, Do), F32),
            jax.ShapeDtypeStruct((NB, 128), F32),
        ],
    )(r, P, batch3d)


def _poolstage(S, C, iso, Wr, Wn, b):
    N = iso.shape[0]
    W = S.shape[2]
    Do = Wr.shape[1]

    def body(s_ref, c_ref, iso_ref, wr_ref, wn_ref, b_ref, r_ref, y_ref):
        cnt = c_ref[0] + c_ref[1]
        m = (s_ref[0] + s_ref[1]) / jnp.clip(cnt[:, 0:1], 1.0, None)
        iso_b = iso_ref[...]
        r_ref[...] = (jnp.dot(m, wr_ref[0:W], preferred_element_type=F32)
                      + jnp.dot(iso_b, wr_ref[W:W + 16],
                                preferred_element_type=F32)
                      + b_ref[...])
        y_ref[...] = (jnp.dot(m, wn_ref[0:W], preferred_element_type=F32)
                      + jnp.dot(iso_b, wn_ref[W:W + 16],
                                preferred_element_type=F32))

    NPp = S.shape[1]
    return pl.pallas_call(
        body, grid=(N // BN,),
        in_specs=[
            pl.BlockSpec((NC, BN, W), lambda i: (0, i, 0)),
            pl.BlockSpec((NC, BN, 16), lambda i: (0, i, 0)),
            pl.BlockSpec((BN, 16), lambda i: (i, 0)),
            pl.BlockSpec((W + 16, Do), lambda i: (0, 0)),
            pl.BlockSpec((W + 16, Do), lambda i: (0, 0)),
            pl.BlockSpec((1, Do), lambda i: (0, 0)),
        ],
        out_specs=[pl.BlockSpec((BN, Do), lambda i: (i, 0))] * 2,
        out_shape=[jax.ShapeDtypeStruct((N, Do), F32)] * 2,
    )(S, C, iso, Wr, Wn, b.reshape(1, -1))


def _head(s1, c1, s2, c2, s3, c3, f1W, f1b, f2W, f2b, f3W, f3b):
    def body(s1_ref, c1_ref, s2_ref, c2_ref, s3_ref, c3_ref,
             w1_ref, b1_ref, w2_ref, b2_ref, w3_ref, b3_ref, o_ref):
        x1 = s1_ref[...] / jnp.clip(c1_ref[...][:, 0:64], 1.0, None)
        x2 = s2_ref[...] / jnp.clip(c2_ref[...][:, 0:64], 1.0, None)
        x3 = s3_ref[...] / jnp.clip(c3_ref[...][:, 0:64], 1.0, None)
        t = (jnp.dot(x1, w1_ref[0:64], preferred_element_type=F32)
             + jnp.dot(x2, w1_ref[64:128], preferred_element_type=F32)
             + jnp.dot(x3, w1_ref[128:192], preferred_element_type=F32)
             + b1_ref[...])
        t = _elu(t)
        t = _elu(jnp.dot(t, w2_ref[...], preferred_element_type=F32)
                 + b2_ref[...])
        o = jnp.dot(t, w3_ref[...], preferred_element_type=F32) + b3_ref[...]
        m = jnp.max(o, axis=1, keepdims=True)
        lse = jnp.log(jnp.sum(jnp.exp(o - m), axis=1, keepdims=True)) + m
        o_ref[...] = o - lse

    full = lambda s: pl.BlockSpec(s, lambda: tuple(0 for _ in s))
    return pl.pallas_call(
        body,
        in_specs=[
            full((NB, 64)), full((NB, 128)),
            full((NB, 64)), full((NB, 128)),
            full((NB, 64)), full((NB, 128)),
            full((192, 64)), full((1, 64)),
            full((64, 32)), full((1, 32)),
            full((32, 10)), full((1, 10)),
        ],
        out_specs=full((NB, 10)),
        out_shape=jax.ShapeDtypeStruct((NB, 10), F32),
    )(s1, c1, s2, c2, s3, c3, f1W, f1b.reshape(1, -1),
      f2W, f2b.reshape(1, -1), f3W, f3b.reshape(1, -1))


def kernel(x, edge_index, batch, a2_src, a2_dst, iso_type_2, edge_index_2,
           batch_2, a3_src, a3_dst, iso_type_3, edge_index_3, batch_3,
           Wr1, Wn1, bc1, Wr2, Wn2, bc2, Wr3, Wn3, bc3, Wr4, Wn4, bc4,
           Wr5, Wn5, bc5, Wr6, Wn6, bc6, Wr7, Wn7, bc7,
           fc1_W, fc1_b, fc2_W, fc2_b, fc3_W, fc3_b):
    i32 = jnp.int32
    e1s, e1d = edge_index[0].astype(i32), edge_index[1].astype(i32)
    e2s, e2d = edge_index_2[0].astype(i32), edge_index_2[1].astype(i32)
    e3s, e3d = edge_index_3[0].astype(i32), edge_index_3[1].astype(i32)
    a2s, a2d = a2_src.astype(i32), a2_dst.astype(i32)
    a3s, a3d = a3_src.astype(i32), a3_dst.astype(i32)
    b1 = batch.astype(i32).reshape(N1_ // BN, 1, BN)
    b2 = batch_2.astype(i32).reshape(N2_ // BN, 1, BN)
    b3 = batch_3.astype(i32).reshape(N3_ // BN, 1, BN)

    r, y = _mm2(x, Wr1, Wn1, bc1)
    (P,) = _segsum(y, e1s, e1d, N1_)
    r, y = _mid(r, P, Wr2, Wn2, bc2)
    (P,) = _segsum(y, e1s, e1d, N1_)
    r, y = _mid(r, P, Wr3, Wn3, bc3)
    (P,) = _segsum(y, e1s, e1d, N1_)
    h, s1, c1 = _last_bpool(r, P, b1)

    S, C = _segsum(h, a2s, a2d, N2_, with_counts=True)
    r, y = _poolstage(S, C, iso_type_2, Wr4, Wn4, bc4)
    (P,) = _segsum(y, e2s, e2d, N2_)
    r, y = _mid(r, P, Wr5, Wn5, bc5)
    (P,) = _segsum(y, e2s, e2d, N2_)
    h2, s2, c2 = _last_bpool(r, P, b2)

    S, C = _segsum(h2, a3s, a3d, N3_, with_counts=True)
    r, y = _poolstage(S, C, iso_type_3, Wr6, Wn6, bc6)
    (P,) = _segsum(y, e3s, e3d, N3_)
    r, y = _mid(r, P, Wr7, Wn7, bc7)
    (P,) = _segsum(y, e3s, e3d, N3_)
    _h3, s3, c3 = _last_bpool(r, P, b3)

    return _head(s1, c1, s2, c2, s3, c3,
                 fc1_W, fc1_b, fc2_W, fc2_b, fc3_W, fc3_b)
